# baseline (speedup 1.0000x reference)
import functools

import jax
import jax.numpy as jnp
from jax import lax
from jax.experimental import pallas as pl
from jax.experimental.pallas import tpu as pltpu
from jax.experimental.pallas import tpu_sc as plsc

NC = 2
NS = 16
NW = NC * NS
LANES = 16


def _make_propagate(n_pad, d, C, SCK, nsc):
    rps = n_pad // NS
    nz = rps // C
    cg_n = d // LANES

    mesh = plsc.VectorSubcoreMesh(
        core_axis_name="c", subcore_axis_name="s", num_cores=NC, num_subcores=NS
    )

    @functools.partial(
        pl.kernel,
        out_type=jax.ShapeDtypeStruct((NC, n_pad, d), jnp.float32),
        mesh=mesh,
        compiler_params=pltpu.CompilerParams(use_tc_tiling_on_sc=(d >= 128)),
        scratch_types=[
            pltpu.VMEM((2, SCK, C), jnp.int32),
            pltpu.VMEM((SCK, C), jnp.float32),
            pltpu.VMEM((C, d), jnp.float32),
            pltpu.VMEM((C, d), jnp.float32),
            pltpu.VMEM_SHARED((n_pad, d), jnp.float32),
            pltpu.SemaphoreType.DMA((2,)),
            pltpu.SemaphoreType.DMA((2,)),
        ],
    )
    def prop(z_hbm, e2_hbm, w_hbm, out_hbm, e2_v, w_v, r0, r1, acc,
             gsem, ssem):
        bufs = (r0, r1)
        cid = lax.axis_index("c")
        sid = lax.axis_index("s")
        wid = cid * NS + sid

        zeros16 = jnp.zeros((LANES,), jnp.float32)

        def zrow(r, carry):
            for cg in range(cg_n):
                r0[r, pl.ds(cg * LANES, LANES)] = zeros16
            return carry

        lax.fori_loop(0, C, zrow, 0)
        base = sid * rps
        for zi in range(nz):
            pltpu.sync_copy(r0, acc.at[pl.ds(base + zi * C, C)])
        plsc.subcore_barrier()

        def scale(k, buf):
            for g in range(C // LANES):
                wg = w_v[k, pl.ds(g * LANES, LANES)]
                for i in range(LANES):
                    ee = g * LANES + i
                    wb = wg.at[jnp.full((LANES,), i, jnp.int32)].get(
                        mode="promise_in_bounds")
                    for cg in range(cg_n):
                        sl = pl.ds(cg * LANES, LANES)
                        buf[ee, sl] = buf[ee, sl] * wb

        def superchunk(j, carry):
            pltpu.sync_copy(e2_hbm.at[wid].at[j], e2_v)
            pltpu.sync_copy(w_hbm.at[wid].at[j], w_v)

            def pair(p, carry2):
                k0 = p * 2
                g0 = pltpu.async_copy(
                    z_hbm.at[e2_v.at[0, k0]], r0, gsem.at[0])
                g1 = pltpu.async_copy(
                    z_hbm.at[e2_v.at[0, k0 + 1]], r1, gsem.at[1])
                g0.wait()
                scale(k0, r0)
                s0 = pltpu.async_copy(
                    r0, acc.at[e2_v.at[1, k0]], ssem.at[0], add=True)
                g1.wait()
                scale(k0 + 1, r1)
                s1 = pltpu.async_copy(
                    r1, acc.at[e2_v.at[1, k0 + 1]], ssem.at[1], add=True)
                s0.wait()
                s1.wait()
                return carry2

            lax.fori_loop(0, SCK // 2, pair, 0)
            return carry

        lax.fori_loop(0, nsc, superchunk, 0)
        plsc.subcore_barrier()

        pltpu.sync_copy(acc.at[pl.ds(base, rps)],
                        out_hbm.at[cid].at[pl.ds(base, rps)])

    return prop


def _matmul(x, w):
    n, din = x.shape
    dout = w.shape[1]
    bm = 1000

    def body(x_ref, w_ref, o_ref):
        o_ref[...] = jnp.dot(x_ref[...], w_ref[...],
                             preferred_element_type=jnp.float32)

    return pl.pallas_call(
        body,
        grid=(n // bm,),
        in_specs=[pl.BlockSpec((bm, din), lambda i: (i, 0)),
                  pl.BlockSpec((din, dout), lambda i: (0, 0))],
        out_specs=pl.BlockSpec((bm, dout), lambda i: (i, 0)),
        out_shape=jax.ShapeDtypeStruct((n, dout), jnp.float32),
    )(x, w)


def _combine_relu_matmul(p, b1, w2, n):
    din = p.shape[2]
    dout = w2.shape[1]
    bm = 1000
    b1r = b1.reshape(1, din)

    def body(p_ref, b_ref, w_ref, o_ref):
        h = jnp.maximum(p_ref[0] + p_ref[1] + b_ref[...], 0.0)
        o_ref[...] = jnp.dot(h, w_ref[...], preferred_element_type=jnp.float32)

    return pl.pallas_call(
        body,
        grid=(n // bm,),
        in_specs=[pl.BlockSpec((2, bm, din), lambda i: (0, i, 0)),
                  pl.BlockSpec((1, din), lambda i: (0, 0)),
                  pl.BlockSpec((din, dout), lambda i: (0, 0))],
        out_specs=pl.BlockSpec((bm, dout), lambda i: (i, 0)),
        out_shape=jax.ShapeDtypeStruct((n, dout), jnp.float32),
    )(p, b1r, w2)


def _combine_bias(q, b2, n):
    d = q.shape[2]
    bm = 1000
    b2r = b2.reshape(1, d)

    def body(q_ref, b_ref, o_ref):
        o_ref[...] = q_ref[0] + q_ref[1] + b_ref[...]

    return pl.pallas_call(
        body,
        grid=(n // bm,),
        in_specs=[pl.BlockSpec((2, bm, d), lambda i: (0, i, 0)),
                  pl.BlockSpec((1, d), lambda i: (0, 0))],
        out_specs=pl.BlockSpec((bm, d), lambda i: (i, 0)),
        out_shape=jax.ShapeDtypeStruct((n, d), jnp.float32),
    )(q, b2r)


def _pack_edges(edge_index, edge_weight, n_pad, C, SCK, nsc):
    e = edge_index.shape[1]
    pad = NW * nsc * SCK * C - e
    src = jnp.concatenate([edge_index[0], jnp.zeros((pad,), jnp.int32)])
    dst = jnp.concatenate(
        [edge_index[1], (jnp.arange(pad, dtype=jnp.int32) % n_pad)])
    wts = jnp.concatenate([edge_weight, jnp.zeros((pad,), jnp.float32)])
    parts = [a.reshape(NW, nsc, 1, SCK, C) for a in (src, dst)]
    e2 = jnp.concatenate(parts, axis=2)
    return e2, wts.reshape(NW, nsc, SCK, C)


def kernel(x, label, mask, edge_index, edge_weight, W1, b1, W2, b2):
    n, d_in = x.shape
    d_h = W1.shape[1]
    d_out = W2.shape[1]
    n_pad = 10240

    e2_l1, w_l1 = _pack_edges(edge_index, edge_weight, n_pad,
                              C=80, SCK=32, nsc=4)
    e2_l2, w_l2 = _pack_edges(edge_index, edge_weight, n_pad,
                              C=128, SCK=82, nsc=1)

    h0 = _matmul(x, W1)
    p1 = _make_propagate(n_pad, d_h, 80, 32, 4)(h0, e2_l1, w_l1)
    h1 = _combine_relu_matmul(p1, b1, W2, n)
    p2 = _make_propagate(n_pad, d_out, 128, 82, 1)(h1, e2_l2, w_l2)
    return _combine_bias(p2, b2, n)


# speedup vs baseline: 1.0176x; 1.0176x over previous
import functools

import jax
import jax.numpy as jnp
from jax import lax
from jax.experimental import pallas as pl
from jax.experimental.pallas import tpu as pltpu
from jax.experimental.pallas import tpu_sc as plsc

NC = 2
NS = 16
NW = NC * NS
LANES = 16


def _make_propagate(n_pad, d, M, C, SCK, nsc):
    rps = n_pad // NS
    nz = rps // C
    cg_n = d // LANES

    mesh = plsc.VectorSubcoreMesh(
        core_axis_name="c", subcore_axis_name="s", num_cores=NC, num_subcores=NS
    )

    @functools.partial(
        pl.kernel,
        out_type=jax.ShapeDtypeStruct((NC, n_pad, d), jnp.float32),
        mesh=mesh,
        compiler_params=pltpu.CompilerParams(use_tc_tiling_on_sc=(d >= 128)),
        scratch_types=[
            pltpu.VMEM((2, SCK, M * C), jnp.int32),
            pltpu.VMEM((SCK, M * C), jnp.float32),
            pltpu.VMEM((M * C, d), jnp.float32),
            pltpu.VMEM_SHARED((n_pad, d), jnp.float32),
            pltpu.SemaphoreType.DMA,
            pltpu.SemaphoreType.DMA,
        ],
    )
    def prop(z_hbm, e2_hbm, w_hbm, out_hbm, e2_v, w_v, rows_v, acc,
             gsem, ssem):
        cid = lax.axis_index("c")
        sid = lax.axis_index("s")
        wid = cid * NS + sid

        zeros16 = jnp.zeros((LANES,), jnp.float32)

        def zrow(r, carry):
            for cg in range(cg_n):
                rows_v[r, pl.ds(cg * LANES, LANES)] = zeros16
            return carry

        lax.fori_loop(0, C, zrow, 0)
        base = sid * rps
        for zi in range(nz):
            pltpu.sync_copy(rows_v.at[pl.ds(0, C)],
                            acc.at[pl.ds(base + zi * C, C)])
        plsc.subcore_barrier()

        def superchunk(j, carry):
            pltpu.sync_copy(e2_hbm.at[wid].at[j], e2_v)
            pltpu.sync_copy(w_hbm.at[wid].at[j], w_v)

            def chunk(k, carry2):
                pltpu.async_copy(
                    z_hbm.at[e2_v.at[0, k]], rows_v, gsem).wait()
                def mscale(s, carry3):
                    for g in range(C // LANES):
                        wg = w_v[k, pl.ds(s * C + g * LANES, LANES)]
                        for i in range(LANES):
                            ee = g * LANES + i
                            wb = wg.at[jnp.full((LANES,), i, jnp.int32)].get(
                                mode="promise_in_bounds")
                            for cg in range(cg_n):
                                sl = pl.ds(cg * LANES, LANES)
                                rows_v[s * C + ee, sl] = (
                                    rows_v[s * C + ee, sl] * wb)
                    return carry3

                lax.fori_loop(0, M, mscale, 0)
                pltpu.async_copy(
                    rows_v, acc.at[e2_v.at[1, k]], ssem, add=True).wait()
                return carry2

            lax.fori_loop(0, SCK, chunk, 0)
            return carry

        lax.fori_loop(0, nsc, superchunk, 0)
        plsc.subcore_barrier()

        pltpu.sync_copy(acc.at[pl.ds(base, rps)],
                        out_hbm.at[cid].at[pl.ds(base, rps)])

    return prop


def _matmul(x, w):
    n, din = x.shape
    dout = w.shape[1]
    bm = 1000

    def body(x_ref, w_ref, o_ref):
        o_ref[...] = jnp.dot(x_ref[...], w_ref[...],
                             preferred_element_type=jnp.float32)

    return pl.pallas_call(
        body,
        grid=(n // bm,),
        in_specs=[pl.BlockSpec((bm, din), lambda i: (i, 0)),
                  pl.BlockSpec((din, dout), lambda i: (0, 0))],
        out_specs=pl.BlockSpec((bm, dout), lambda i: (i, 0)),
        out_shape=jax.ShapeDtypeStruct((n, dout), jnp.float32),
    )(x, w)


def _combine_relu_matmul(p, b1, w2, n):
    din = p.shape[2]
    dout = w2.shape[1]
    bm = 1000
    b1r = b1.reshape(1, din)

    def body(p_ref, b_ref, w_ref, o_ref):
        h = jnp.maximum(p_ref[0] + p_ref[1] + b_ref[...], 0.0)
        o_ref[...] = jnp.dot(h, w_ref[...], preferred_element_type=jnp.float32)

    return pl.pallas_call(
        body,
        grid=(n // bm,),
        in_specs=[pl.BlockSpec((2, bm, din), lambda i: (0, i, 0)),
                  pl.BlockSpec((1, din), lambda i: (0, 0)),
                  pl.BlockSpec((din, dout), lambda i: (0, 0))],
        out_specs=pl.BlockSpec((bm, dout), lambda i: (i, 0)),
        out_shape=jax.ShapeDtypeStruct((n, dout), jnp.float32),
    )(p, b1r, w2)


def _combine_bias(q, b2, n):
    d = q.shape[2]
    bm = 1000
    b2r = b2.reshape(1, d)

    def body(q_ref, b_ref, o_ref):
        o_ref[...] = q_ref[0] + q_ref[1] + b_ref[...]

    return pl.pallas_call(
        body,
        grid=(n // bm,),
        in_specs=[pl.BlockSpec((2, bm, d), lambda i: (0, i, 0)),
                  pl.BlockSpec((1, d), lambda i: (0, 0))],
        out_specs=pl.BlockSpec((bm, d), lambda i: (i, 0)),
        out_shape=jax.ShapeDtypeStruct((n, d), jnp.float32),
    )(q, b2r)


def _pack_edges(edge_index, edge_weight, n_pad, M, C, SCK, nsc):
    e = edge_index.shape[1]
    pad = NW * nsc * SCK * M * C - e
    src = jnp.concatenate([edge_index[0], jnp.zeros((pad,), jnp.int32)])
    dst = jnp.concatenate(
        [edge_index[1], (jnp.arange(pad, dtype=jnp.int32) % n_pad)])
    wts = jnp.concatenate([edge_weight, jnp.zeros((pad,), jnp.float32)])
    parts = [a.reshape(NW, nsc, 1, SCK, M * C) for a in (src, dst)]
    e2 = jnp.concatenate(parts, axis=2)
    return e2, wts.reshape(NW, nsc, SCK, M * C)


def kernel(x, label, mask, edge_index, edge_weight, W1, b1, W2, b2):
    n, d_in = x.shape
    d_h = W1.shape[1]
    d_out = W2.shape[1]
    n_pad = 10240

    e2_l1, w_l1 = _pack_edges(edge_index, edge_weight, n_pad,
                              M=1, C=128, SCK=27, nsc=3)
    e2_l2, w_l2 = _pack_edges(edge_index, edge_weight, n_pad,
                              M=8, C=128, SCK=10, nsc=1)

    h0 = _matmul(x, W1)
    p1 = _make_propagate(n_pad, d_h, 1, 128, 27, 3)(h0, e2_l1, w_l1)
    h1 = _combine_relu_matmul(p1, b1, W2, n)
    p2 = _make_propagate(n_pad, d_out, 8, 128, 10, 1)(h1, e2_l2, w_l2)
    return _combine_bias(p2, b2, n)


# speedup vs baseline: 1.9234x; 1.8900x over previous
import functools

import jax
import jax.numpy as jnp
from jax import lax
from jax.experimental import pallas as pl
from jax.experimental.pallas import tpu as pltpu
from jax.experimental.pallas import tpu_sc as plsc

NC = 2
NS = 16
NW = NC * NS
LANES = 16


def _make_propagate(n_pad, d, M, C, SCK, nsc):
    rps = n_pad // NS
    nz = rps // C
    cg_n = d // LANES

    mesh = plsc.VectorSubcoreMesh(
        core_axis_name="c", subcore_axis_name="s", num_cores=NC, num_subcores=NS
    )

    @functools.partial(
        pl.kernel,
        out_type=jax.ShapeDtypeStruct((NC, n_pad, d), jnp.float32),
        mesh=mesh,
        compiler_params=pltpu.CompilerParams(use_tc_tiling_on_sc=(d >= 128)),
        scratch_types=[
            pltpu.VMEM((2, SCK, M * C), jnp.int32),
            pltpu.VMEM((SCK, M * C), jnp.float32),
            pltpu.VMEM((M * C, d), jnp.float32),
            pltpu.VMEM_SHARED((n_pad, d), jnp.float32),
            pltpu.SemaphoreType.DMA,
            pltpu.SemaphoreType.DMA,
        ],
    )
    def prop(z_hbm, e2_hbm, w_hbm, out_hbm, e2_v, w_v, rows_v, acc,
             gsem, ssem):
        cid = lax.axis_index("c")
        sid = lax.axis_index("s")
        wid = cid * NS + sid

        zeros16 = jnp.zeros((LANES,), jnp.float32)

        def zrow(r, carry):
            for cg in range(cg_n):
                rows_v[r, pl.ds(cg * LANES, LANES)] = zeros16
            return carry

        lax.fori_loop(0, C, zrow, 0)
        base = sid * rps
        for zi in range(nz):
            pltpu.sync_copy(rows_v.at[pl.ds(0, C)],
                            acc.at[pl.ds(base + zi * C, C)])
        plsc.subcore_barrier()

        def superchunk(j, carry):
            pltpu.sync_copy(e2_hbm.at[wid].at[j], e2_v)
            pltpu.sync_copy(w_hbm.at[wid].at[j], w_v)

            def chunk(k, carry2):
                pltpu.async_copy(
                    z_hbm.at[e2_v.at[0, k]], rows_v, gsem).wait()
                def mscale(s, carry3):
                    for g in range(C // LANES):
                        wg = w_v[k, pl.ds(s * C + g * LANES, LANES)]
                        for i in range(LANES):
                            ee = g * LANES + i
                            wb = wg.at[jnp.full((LANES,), i, jnp.int32)].get(
                                mode="promise_in_bounds")
                            for cg in range(cg_n):
                                sl = pl.ds(cg * LANES, LANES)
                                rows_v[s * C + ee, sl] = (
                                    rows_v[s * C + ee, sl] * wb)
                    return carry3

                lax.fori_loop(0, M, mscale, 0)
                pltpu.async_copy(
                    rows_v, acc.at[e2_v.at[1, k]], ssem, add=True).wait()
                return carry2

            lax.fori_loop(0, SCK, chunk, 0)
            return carry

        lax.fori_loop(0, nsc, superchunk, 0)
        plsc.subcore_barrier()

        pltpu.sync_copy(acc.at[pl.ds(base, rps)],
                        out_hbm.at[cid].at[pl.ds(base, rps)])

    return prop


def _matmul(x, w):
    n, din = x.shape
    dout = w.shape[1]
    bm = 1000

    def body(x_ref, w_ref, o_ref):
        o_ref[...] = jnp.dot(x_ref[...], w_ref[...],
                             preferred_element_type=jnp.float32)

    return pl.pallas_call(
        body,
        grid=(n // bm,),
        in_specs=[pl.BlockSpec((bm, din), lambda i: (i, 0)),
                  pl.BlockSpec((din, dout), lambda i: (0, 0))],
        out_specs=pl.BlockSpec((bm, dout), lambda i: (i, 0)),
        out_shape=jax.ShapeDtypeStruct((n, dout), jnp.float32),
    )(x, w)


def _combine_relu_matmul(p, b1, w2, n):
    din = p.shape[2]
    dout = w2.shape[1]
    bm = 1000
    b1r = b1.reshape(1, din)

    def body(p_ref, b_ref, w_ref, o_ref):
        h = jnp.maximum(p_ref[0] + p_ref[1] + b_ref[...], 0.0)
        o_ref[...] = jnp.dot(h, w_ref[...], preferred_element_type=jnp.float32)

    return pl.pallas_call(
        body,
        grid=(n // bm,),
        in_specs=[pl.BlockSpec((2, bm, din), lambda i: (0, i, 0)),
                  pl.BlockSpec((1, din), lambda i: (0, 0)),
                  pl.BlockSpec((din, dout), lambda i: (0, 0))],
        out_specs=pl.BlockSpec((bm, dout), lambda i: (i, 0)),
        out_shape=jax.ShapeDtypeStruct((n, dout), jnp.float32),
    )(p, b1r, w2)


def _combine_bias(q, b2, n):
    d = q.shape[2]
    bm = 1000
    b2r = b2.reshape(1, d)

    def body(q_ref, b_ref, o_ref):
        o_ref[...] = q_ref[0] + q_ref[1] + b_ref[...]

    return pl.pallas_call(
        body,
        grid=(n // bm,),
        in_specs=[pl.BlockSpec((2, bm, d), lambda i: (0, i, 0)),
                  pl.BlockSpec((1, d), lambda i: (0, 0))],
        out_specs=pl.BlockSpec((bm, d), lambda i: (i, 0)),
        out_shape=jax.ShapeDtypeStruct((n, d), jnp.float32),
    )(q, b2r)


def _pack_edges(edge_index, edge_weight, n_pad, M, C, SCK, nsc):
    e = edge_index.shape[1]
    pad = NW * nsc * SCK * M * C - e
    src = jnp.concatenate([edge_index[0], jnp.zeros((pad,), jnp.int32)])
    dst = jnp.concatenate(
        [edge_index[1], (jnp.arange(pad, dtype=jnp.int32) % n_pad)])
    wts = jnp.concatenate([edge_weight, jnp.zeros((pad,), jnp.float32)])
    parts = [a.reshape(NW, nsc, 1, SCK, M * C) for a in (src, dst)]
    e2 = jnp.concatenate(parts, axis=2)
    return e2, wts.reshape(NW, nsc, SCK, M * C)


def kernel(x, label, mask, edge_index, edge_weight, W1, b1, W2, b2):
    n, d_in = x.shape
    d_h = W1.shape[1]
    d_out = W2.shape[1]
    n_pad = 10240

    e2_l1, w_l1 = _pack_edges(edge_index, edge_weight, n_pad,
                              M=1, C=80, SCK=25, nsc=5)
    e2_l2, w_l2 = _pack_edges(edge_index, edge_weight, n_pad,
                              M=8, C=128, SCK=10, nsc=1)

    h0 = _matmul(x, W1)
    p1 = _make_propagate(n_pad, d_h, 1, 80, 25, 5)(h0, e2_l1, w_l1)
    h1 = _combine_relu_matmul(p1, b1, W2, n)
    p2 = _make_propagate(n_pad, d_out, 8, 128, 10, 1)(h1, e2_l2, w_l2)
    return _combine_bias(p2, b2, n)


# speedup vs baseline: 2.2835x; 1.1873x over previous
import functools

import jax
import jax.numpy as jnp
from jax import lax
from jax.experimental import pallas as pl
from jax.experimental.pallas import tpu as pltpu
from jax.experimental.pallas import tpu_sc as plsc

NC = 2
NS = 16
NW = NC * NS
LANES = 16


def _make_propagate(n_pad, d, M, C, SCK, nsc):
    rps = n_pad // NS
    nz = rps // C
    cg_n = d // LANES

    mesh = plsc.VectorSubcoreMesh(
        core_axis_name="c", subcore_axis_name="s", num_cores=NC, num_subcores=NS
    )

    @functools.partial(
        pl.kernel,
        out_type=jax.ShapeDtypeStruct((NC, n_pad, d), jnp.float32),
        mesh=mesh,
        compiler_params=pltpu.CompilerParams(use_tc_tiling_on_sc=(d >= 128)),
        scratch_types=[
            pltpu.VMEM((2, SCK, M * C), jnp.int32),
            pltpu.VMEM((SCK, M * C), jnp.float32),
            pltpu.VMEM((M * C, d), jnp.float32),
            pltpu.VMEM_SHARED((n_pad, d), jnp.float32),
            pltpu.SemaphoreType.DMA,
            pltpu.SemaphoreType.DMA,
        ],
    )
    def prop(z_hbm, e2_hbm, w_hbm, out_hbm, e2_v, w_v, rows_v, acc,
             gsem, ssem):
        cid = lax.axis_index("c")
        sid = lax.axis_index("s")
        wid = cid * NS + sid

        zeros16 = jnp.zeros((LANES,), jnp.float32)

        def zrow(r, carry):
            for cg in range(cg_n):
                rows_v[r, pl.ds(cg * LANES, LANES)] = zeros16
            return carry

        lax.fori_loop(0, C, zrow, 0)
        base = sid * rps
        for zi in range(nz):
            pltpu.sync_copy(rows_v.at[pl.ds(0, C)],
                            acc.at[pl.ds(base + zi * C, C)])
        plsc.subcore_barrier()

        def superchunk(j, carry):
            pltpu.sync_copy(e2_hbm.at[wid].at[j], e2_v)
            pltpu.sync_copy(w_hbm.at[wid].at[j], w_v)

            def chunk(k, carry2):
                pltpu.async_copy(
                    z_hbm.at[e2_v.at[0, k]], rows_v, gsem).wait()
                def mscale(s, carry3):
                    for g in range(C // LANES):
                        wg = w_v[k, pl.ds(s * C + g * LANES, LANES)]
                        for i in range(LANES):
                            ee = g * LANES + i
                            wb = wg.at[jnp.full((LANES,), i, jnp.int32)].get(
                                mode="promise_in_bounds")
                            for cg in range(cg_n):
                                sl = pl.ds(cg * LANES, LANES)
                                rows_v[s * C + ee, sl] = (
                                    rows_v[s * C + ee, sl] * wb)
                    return carry3

                lax.fori_loop(0, M, mscale, 0)
                pltpu.async_copy(
                    rows_v, acc.at[e2_v.at[1, k]], ssem, add=True).wait()
                return carry2

            lax.fori_loop(0, SCK, chunk, 0)
            return carry

        lax.fori_loop(0, nsc, superchunk, 0)
        plsc.subcore_barrier()

        pltpu.sync_copy(acc.at[pl.ds(base, rps)],
                        out_hbm.at[cid].at[pl.ds(base, rps)])

    return prop


def _matmul(x, w):
    n, din = x.shape
    dout = w.shape[1]
    bm = 1000

    def body(x_ref, w_ref, o_ref):
        o_ref[...] = jnp.dot(x_ref[...], w_ref[...],
                             preferred_element_type=jnp.float32)

    return pl.pallas_call(
        body,
        grid=(n // bm,),
        in_specs=[pl.BlockSpec((bm, din), lambda i: (i, 0)),
                  pl.BlockSpec((din, dout), lambda i: (0, 0))],
        out_specs=pl.BlockSpec((bm, dout), lambda i: (i, 0)),
        out_shape=jax.ShapeDtypeStruct((n, dout), jnp.float32),
    )(x, w)


def _combine_relu_matmul(p, b1, w2, n):
    din = p.shape[2]
    dout = w2.shape[1]
    bm = 1000
    b1r = b1.reshape(1, din)

    def body(p_ref, b_ref, w_ref, o_ref):
        h = jnp.maximum(p_ref[0] + p_ref[1] + b_ref[...], 0.0)
        o_ref[...] = jnp.dot(h, w_ref[...], preferred_element_type=jnp.float32)

    return pl.pallas_call(
        body,
        grid=(n // bm,),
        in_specs=[pl.BlockSpec((2, bm, din), lambda i: (0, i, 0)),
                  pl.BlockSpec((1, din), lambda i: (0, 0)),
                  pl.BlockSpec((din, dout), lambda i: (0, 0))],
        out_specs=pl.BlockSpec((bm, dout), lambda i: (i, 0)),
        out_shape=jax.ShapeDtypeStruct((n, dout), jnp.float32),
    )(p, b1r, w2)


def _combine_bias(q, b2, n):
    d = q.shape[2]
    bm = 1000
    b2r = b2.reshape(1, d)

    def body(q_ref, b_ref, o_ref):
        o_ref[...] = q_ref[0] + q_ref[1] + b_ref[...]

    return pl.pallas_call(
        body,
        grid=(n // bm,),
        in_specs=[pl.BlockSpec((2, bm, d), lambda i: (0, i, 0)),
                  pl.BlockSpec((1, d), lambda i: (0, 0))],
        out_specs=pl.BlockSpec((bm, d), lambda i: (i, 0)),
        out_shape=jax.ShapeDtypeStruct((n, d), jnp.float32),
    )(q, b2r)


def _pack_edges(edge_index, edge_weight, n, n_pad, M, C, SCK, nsc):
    e = edge_index.shape[1]
    pad = NW * nsc * SCK * M * C - e
    src = jnp.concatenate(
        [edge_index[0], (jnp.arange(pad, dtype=jnp.int32) * 7) % n])
    dst = jnp.concatenate(
        [edge_index[1], (jnp.arange(pad, dtype=jnp.int32) % n_pad)])
    wts = jnp.concatenate([edge_weight, jnp.zeros((pad,), jnp.float32)])
    parts = [a.reshape(NW, nsc, 1, SCK, M * C) for a in (src, dst)]
    e2 = jnp.concatenate(parts, axis=2)
    return e2, wts.reshape(NW, nsc, SCK, M * C)


def kernel(x, label, mask, edge_index, edge_weight, W1, b1, W2, b2):
    n, d_in = x.shape
    d_h = W1.shape[1]
    d_out = W2.shape[1]
    n_pad = 10240

    e2_l1, w_l1 = _pack_edges(edge_index, edge_weight, n, n_pad,
                              M=1, C=128, SCK=27, nsc=3)
    e2_l2, w_l2 = _pack_edges(edge_index, edge_weight, n, n_pad,
                              M=8, C=128, SCK=10, nsc=1)

    h0 = _matmul(x, W1)
    p1 = _make_propagate(n_pad, d_h, 1, 128, 27, 3)(h0, e2_l1, w_l1)
    h1 = _combine_relu_matmul(p1, b1, W2, n)
    p2 = _make_propagate(n_pad, d_out, 8, 128, 10, 1)(h1, e2_l2, w_l2)
    return _combine_bias(p2, b2, n)


# speedup vs baseline: 2.3026x; 1.0083x over previous
import functools

import jax
import jax.numpy as jnp
from jax import lax
from jax.experimental import pallas as pl
from jax.experimental.pallas import tpu as pltpu
from jax.experimental.pallas import tpu_sc as plsc

NC = 2
NS = 16
NW = NC * NS
LANES = 16


def _make_propagate(n_pad, d, M, C, SCK, nsc):
    rps = n_pad // NS
    nz = rps // C
    cg_n = d // LANES

    mesh = plsc.VectorSubcoreMesh(
        core_axis_name="c", subcore_axis_name="s", num_cores=NC, num_subcores=NS
    )

    @functools.partial(
        pl.kernel,
        out_type=jax.ShapeDtypeStruct((NC, n_pad, d), jnp.float32),
        mesh=mesh,
        compiler_params=pltpu.CompilerParams(use_tc_tiling_on_sc=(d >= 128)),
        scratch_types=[
            pltpu.VMEM((2, SCK, M * C), jnp.int32),
            pltpu.VMEM((SCK, M * C), jnp.float32),
            pltpu.VMEM((M * C, d), jnp.float32),
            pltpu.VMEM_SHARED((n_pad, d), jnp.float32),
            pltpu.SemaphoreType.DMA,
            pltpu.SemaphoreType.DMA,
        ],
    )
    def prop(z_hbm, e2_hbm, w_hbm, out_hbm, e2_v, w_v, rows_v, acc,
             gsem, ssem):
        cid = lax.axis_index("c")
        sid = lax.axis_index("s")
        wid = cid * NS + sid

        zeros16 = jnp.zeros((LANES,), jnp.float32)

        def zrow(r, carry):
            for cg in range(cg_n):
                rows_v[r, pl.ds(cg * LANES, LANES)] = zeros16
            return carry

        lax.fori_loop(0, C, zrow, 0)
        base = sid * rps
        for zi in range(nz):
            pltpu.sync_copy(rows_v.at[pl.ds(0, C)],
                            acc.at[pl.ds(base + zi * C, C)])
        plsc.subcore_barrier()

        def superchunk(j, carry):
            pltpu.sync_copy(e2_hbm.at[wid].at[j], e2_v)
            pltpu.sync_copy(w_hbm.at[wid].at[j], w_v)

            def chunk(k, carry2):
                pltpu.async_copy(
                    z_hbm.at[e2_v.at[0, k]], rows_v, gsem).wait()
                def mscale(s, carry3):
                    for g in range(C // LANES):
                        wg = w_v[k, pl.ds(s * C + g * LANES, LANES)]
                        for i in range(LANES):
                            ee = g * LANES + i
                            wb = wg.at[jnp.full((LANES,), i, jnp.int32)].get(
                                mode="promise_in_bounds")
                            for cg in range(cg_n):
                                sl = pl.ds(cg * LANES, LANES)
                                rows_v[s * C + ee, sl] = (
                                    rows_v[s * C + ee, sl] * wb)
                    return carry3

                lax.fori_loop(0, M, mscale, 0)
                pltpu.async_copy(
                    rows_v, acc.at[e2_v.at[1, k]], ssem, add=True).wait()
                return carry2

            lax.fori_loop(0, SCK, chunk, 0)
            return carry

        lax.fori_loop(0, nsc, superchunk, 0)
        plsc.subcore_barrier()

        pltpu.sync_copy(acc.at[pl.ds(base, rps)],
                        out_hbm.at[cid].at[pl.ds(base, rps)])

    return prop


def _matmul(x, w):
    n, din = x.shape
    dout = w.shape[1]
    bm = 1000

    def body(x_ref, w_ref, o_ref):
        o_ref[...] = jnp.dot(x_ref[...], w_ref[...],
                             preferred_element_type=jnp.float32)

    return pl.pallas_call(
        body,
        grid=(n // bm,),
        in_specs=[pl.BlockSpec((bm, din), lambda i: (i, 0)),
                  pl.BlockSpec((din, dout), lambda i: (0, 0))],
        out_specs=pl.BlockSpec((bm, dout), lambda i: (i, 0)),
        out_shape=jax.ShapeDtypeStruct((n, dout), jnp.float32),
    )(x, w)


def _combine_relu_matmul(p, b1, w2, n):
    din = p.shape[2]
    dout = w2.shape[1]
    bm = 1000
    b1r = b1.reshape(1, din)

    def body(p_ref, b_ref, w_ref, o_ref):
        h = jnp.maximum(p_ref[0] + p_ref[1] + b_ref[...], 0.0)
        o_ref[...] = jnp.dot(h, w_ref[...], preferred_element_type=jnp.float32)

    return pl.pallas_call(
        body,
        grid=(n // bm,),
        in_specs=[pl.BlockSpec((2, bm, din), lambda i: (0, i, 0)),
                  pl.BlockSpec((1, din), lambda i: (0, 0)),
                  pl.BlockSpec((din, dout), lambda i: (0, 0))],
        out_specs=pl.BlockSpec((bm, dout), lambda i: (i, 0)),
        out_shape=jax.ShapeDtypeStruct((n, dout), jnp.float32),
    )(p, b1r, w2)


def _combine_bias(q, b2, n):
    d = q.shape[2]
    bm = 1000
    b2r = b2.reshape(1, d)

    def body(q_ref, b_ref, o_ref):
        o_ref[...] = q_ref[0] + q_ref[1] + b_ref[...]

    return pl.pallas_call(
        body,
        grid=(n // bm,),
        in_specs=[pl.BlockSpec((2, bm, d), lambda i: (0, i, 0)),
                  pl.BlockSpec((1, d), lambda i: (0, 0))],
        out_specs=pl.BlockSpec((bm, d), lambda i: (i, 0)),
        out_shape=jax.ShapeDtypeStruct((n, d), jnp.float32),
    )(q, b2r)


def _pack_edges(edge_index, edge_weight, n, n_pad, M, C, SCK, nsc):
    e = edge_index.shape[1]
    pad = NW * nsc * SCK * M * C - e
    src = jnp.concatenate(
        [edge_index[0], (jnp.arange(pad, dtype=jnp.int32) * 7) % n])
    dst = jnp.concatenate(
        [edge_index[1], (jnp.arange(pad, dtype=jnp.int32) % n_pad)])
    wts = jnp.concatenate([edge_weight, jnp.zeros((pad,), jnp.float32)])
    parts = [a.reshape(NW, nsc, 1, SCK, M * C) for a in (src, dst)]
    e2 = jnp.concatenate(parts, axis=2)
    return e2, wts.reshape(NW, nsc, SCK, M * C)


def kernel(x, label, mask, edge_index, edge_weight, W1, b1, W2, b2):
    n, d_in = x.shape
    d_h = W1.shape[1]
    d_out = W2.shape[1]
    n_pad = 10240

    e2_l1, w_l1 = _pack_edges(edge_index, edge_weight, n, n_pad,
                              M=1, C=128, SCK=27, nsc=3)
    e2_l2, w_l2 = _pack_edges(edge_index, edge_weight, n, n_pad,
                              M=16, C=128, SCK=5, nsc=1)

    h0 = _matmul(x, W1)
    p1 = _make_propagate(n_pad, d_h, 1, 128, 27, 3)(h0, e2_l1, w_l1)
    h1 = _combine_relu_matmul(p1, b1, W2, n)
    p2 = _make_propagate(n_pad, d_out, 16, 128, 5, 1)(h1, e2_l2, w_l2)
    return _combine_bias(p2, b2, n)
